# Initial kernel scaffold; baseline (speedup 1.0000x reference)
import jax
import jax.numpy as jnp
from jax.experimental import pallas as pl


def kernel(h, x, edge_index, W1, b1, W2, b2, Wc, bc, Wn, bn, gamma, beta, crw):
    raise NotImplementedError("write your pallas kernel here")


# speedup vs baseline: 1.9251x; 1.9251x over previous
import functools

import jax
import jax.numpy as jnp
from jax import lax
from jax.experimental import pallas as pl
from jax.experimental.pallas import tpu as pltpu
from jax.experimental.pallas import tpu_sc as plsc

F32 = jnp.float32
NC, NS, L = 2, 16, 16
BR = 256


def _pre_body(h_ref, w1a_ref, w1b_ref, b1_ref, w2_ref, wct_ref, b2_ref,
              bc_ref, crw_ref, ha_out, hb_out, w2c_out, c_out, crw_out):
    hblk = h_ref[...]
    ha_out[...] = jnp.dot(hblk, w1a_ref[...], precision=lax.Precision.HIGHEST,
                          preferred_element_type=F32) + b1_ref[...]
    hb_out[...] = jnp.dot(hblk, w1b_ref[...], precision=lax.Precision.HIGHEST,
                          preferred_element_type=F32)
    wct = wct_ref[...]
    w2c_out[...] = lax.dot_general(wct, w2_ref[...], (((1,), (1,)), ((), ())),
                                   precision=lax.Precision.HIGHEST,
                                   preferred_element_type=F32)
    cval = lax.dot_general(b2_ref[...], wct, (((1,), (1,)), ((), ())),
                           precision=lax.Precision.HIGHEST,
                           preferred_element_type=F32)
    z = jnp.zeros_like(wct)
    c_out[...] = z + (cval[0, 0] + bc_ref[0, 0])
    crw_out[...] = z + crw_ref[0, 0]


def _post_body(h_ref, s0_ref, s1_ref, d0_ref, d1_ref, x_ref, w2_ref, b2_ref,
               wna_ref, wnb_ref, bn_ref, g_ref, b_ref, crw_ref,
               ho_out, xo_out):
    s = s0_ref[...] + s1_ref[...]
    dsum = d0_ref[...] + d1_ref[...]
    cnt = dsum[:, 3:4]
    agg = jnp.dot(s, w2_ref[...], precision=lax.Precision.HIGHEST,
                  preferred_element_type=F32) + cnt * b2_ref[...]
    hblk = h_ref[...]
    hn = jnp.dot(hblk, wna_ref[...], precision=lax.Precision.HIGHEST,
                 preferred_element_type=F32)
    hn = hn + jnp.dot(agg, wnb_ref[...], precision=lax.Precision.HIGHEST,
                      preferred_element_type=F32) + bn_ref[...]
    hn = jnp.maximum(hn, 0.0)
    y = hblk + hn
    mu = jnp.mean(y, axis=-1, keepdims=True)
    var = jnp.mean((y - mu) ** 2, axis=-1, keepdims=True)
    ho_out[...] = (y - mu) * lax.rsqrt(var + 1e-5) * g_ref[...] + b_ref[...]
    xo_out[...] = x_ref[...] + crw_ref[0, 0] * dsum


def _sc_body(ha_hbm, hb_hbm, x16_hbm, row_hbm, col_hbm, w1c_hbm, w2c_hbm,
             c_hbm, zs_hbm, zd_hbm, s_out, d_out,
             idxr, idxc, ha, hb, xr16, xc16, d2b, dotb, sbuf, cub,
             w1cb, w2cb, cvb, s_sh, d_sh, *, ssh, e_per_tile, ch, dw):
    cid = lax.axis_index("c")
    sid = lax.axis_index("s")
    wid = cid * NS + sid
    rows_per_tile = ssh // NS
    nchunks = e_per_tile // ch

    pltpu.sync_copy(w1c_hbm, w1cb)
    pltpu.sync_copy(w2c_hbm.at[0], w2cb)
    pltpu.sync_copy(c_hbm.at[0, pl.ds(0, L)], cvb)

    pltpu.sync_copy(zs_hbm, s_sh.at[pl.ds(sid * rows_per_tile, rows_per_tile)])
    pltpu.sync_copy(zd_hbm, d_sh.at[pl.ds(sid * rows_per_tile, rows_per_tile)])

    lanes = lax.iota(jnp.int32, L)

    plsc.subcore_barrier()

    w1v = [w1cb[pl.ds(L * j, L)] for j in range(8)]
    w2v = [w2cb[pl.ds(L * j, L)] for j in range(8)]
    cvec = cvb[...]
    m15 = lanes == 15

    def _chunk(ci, _):
        base = wid * e_per_tile + ci * ch
        pltpu.sync_copy(row_hbm.at[pl.ds(base, ch)], idxr)
        pltpu.sync_copy(col_hbm.at[pl.ds(base, ch)], idxc)
        pltpu.sync_copy(ha_hbm.at[idxr], ha)
        pltpu.sync_copy(hb_hbm.at[idxc], hb)
        pltpu.sync_copy(x16_hbm.at[idxr], xr16)
        pltpu.sync_copy(x16_hbm.at[idxc], xc16)

        def _edge(e, _):
            ef = jnp.full((L,), e, jnp.int32)
            rel = xr16[e, :] - xc16[e, :]
            d2c = plsc.cumsum(rel * rel)
            plsc.store_scatter(d2b, [ef], d2c, mask=m15)
            d2s = plsc.load_gather(d2b, [ef])
            acc = jnp.zeros((L,), F32)
            for j in range(8):
                sl = pl.ds(L * j, L)
                t = jnp.maximum(ha[e, sl] + hb[e, sl] + d2s * w1v[j], 0.0)
                acc = acc + t * w2v[j]
                ha[e, sl] = t
            cs = plsc.cumsum(acc)
            plsc.store_scatter(dotb, [ef], cs, mask=m15)
            return 0
        lax.fori_loop(0, ch, _edge, 0)

        for g in range(ch // L):
            sl = pl.ds(L * g, L)
            zv = jnp.maximum(dotb[sl] + cvec, 0.0)
            e2 = jnp.exp(zv + zv)
            sbuf[sl] = 1.0 - 2.0 / (e2 + 1.0)

        def _cu(e, _):
            ef = jnp.full((L,), e, jnp.int32)
            rel = xr16[e, :] - xc16[e, :]
            ssp = plsc.load_gather(sbuf, [ef])
            cub[e, :] = jnp.where(lanes < 3, rel * ssp,
                                  jnp.where(lanes == 3, 1.0, 0.0))
            return 0
        lax.fori_loop(0, ch, _cu, 0)

        pltpu.sync_copy(ha, s_sh.at[idxr], add=True)
        pltpu.sync_copy(cub, d_sh.at[idxr], add=True)
        return 0

    lax.fori_loop(0, nchunks, _chunk, 0)

    plsc.subcore_barrier()

    ofs = sid * rows_per_tile
    pltpu.sync_copy(s_sh.at[pl.ds(ofs, rows_per_tile)],
                    s_out.at[cid, pl.ds(ofs, rows_per_tile)])
    pltpu.sync_copy(d_sh.at[pl.ds(ofs, rows_per_tile)],
                    d_out.at[cid, pl.ds(ofs, rows_per_tile)])


def kernel(h, x, edge_index, W1, b1, W2, b2, Wc, bc, Wn, bn, gamma, beta, crw):
    N, D = h.shape
    E = edge_index.shape[1]
    assert N == 10000 and D == 128 and E == 320000
    Npad = 10240
    CH = 64
    EPT = 10240
    Epad = EPT * NC * NS
    SSH = 10112
    DW = 16

    hp = jnp.pad(h, ((0, Npad - N), (0, 0)))
    xp16 = jnp.pad(x, ((0, Npad - N), (0, DW - 3)))
    pad_idx = jnp.full((Epad - E,), N, jnp.int32)
    row = jnp.concatenate([edge_index[0], pad_idx])
    col = jnp.concatenate([edge_index[1], pad_idx])
    W1a, W1b, w1c = W1[:D], W1[D:2 * D], W1[2 * D]
    WnA, WnB = Wn[:D], Wn[D:]
    b1r = b1.reshape(1, D)
    b2r = b2.reshape(1, D)
    bnr = bn.reshape(1, D)
    gr = gamma.reshape(1, D)
    btr = beta.reshape(1, D)
    wct = Wc.reshape(1, D)
    bcr = bc.reshape(1, 1)
    crwr = crw.reshape(1, 1)
    zs = jnp.zeros((SSH // NS, D), F32)
    zd = jnp.zeros((SSH // NS, DW), F32)

    grid = (Npad // BR,)
    full = lambda: pl.BlockSpec(index_map=lambda i: (0, 0))
    rblk = lambda w: pl.BlockSpec((BR, w), lambda i: (i, 0))

    ha_full, hb_full, w2cr, cr, crwrow = pl.pallas_call(
        _pre_body,
        grid=grid,
        in_specs=[rblk(D), full(), full(), full(), full(), full(), full(),
                  full(), full()],
        out_specs=[rblk(D), rblk(D), full(), full(), full()],
        out_shape=[jax.ShapeDtypeStruct((Npad, D), F32),
                   jax.ShapeDtypeStruct((Npad, D), F32),
                   jax.ShapeDtypeStruct((1, D), F32),
                   jax.ShapeDtypeStruct((1, D), F32),
                   jax.ShapeDtypeStruct((1, D), F32)],
    )(hp, W1a, W1b, b1r, W2, wct, b2r, bcr, crwr)

    mesh = plsc.VectorSubcoreMesh(core_axis_name="c", subcore_axis_name="s",
                                  num_cores=NC, num_subcores=NS)
    sc = pl.kernel(
        functools.partial(_sc_body, ssh=SSH, e_per_tile=EPT, ch=CH, dw=DW),
        out_type=[jax.ShapeDtypeStruct((NC, Npad, D), F32),
                  jax.ShapeDtypeStruct((NC, Npad, DW), F32)],
        mesh=mesh,
        compiler_params=pltpu.CompilerParams(use_tc_tiling_on_sc=False,
                                             needs_layout_passes=False),
        scratch_types=[
            pltpu.VMEM((CH,), jnp.int32),
            pltpu.VMEM((CH,), jnp.int32),
            pltpu.VMEM((CH, D), F32),
            pltpu.VMEM((CH, D), F32),
            pltpu.VMEM((CH, DW), F32),
            pltpu.VMEM((CH, DW), F32),
            pltpu.VMEM((CH,), F32),
            pltpu.VMEM((CH,), F32),
            pltpu.VMEM((CH,), F32),
            pltpu.VMEM((CH, DW), F32),
            pltpu.VMEM((D,), F32),
            pltpu.VMEM((D,), F32),
            pltpu.VMEM((L,), F32),
            pltpu.VMEM_SHARED((SSH, D), F32),
            pltpu.VMEM_SHARED((SSH, DW), F32),
        ],
    )
    s_parts, d_parts = sc(ha_full, hb_full, xp16, row, col, w1c, w2cr, cr,
                          zs, zd)

    ho, xo = pl.pallas_call(
        _post_body,
        grid=grid,
        in_specs=[rblk(D), rblk(D), rblk(D), rblk(DW), rblk(DW), rblk(DW),
                  full(), full(), full(), full(), full(), full(), full(),
                  full()],
        out_specs=[rblk(D), rblk(DW)],
        out_shape=[jax.ShapeDtypeStruct((Npad, D), F32),
                   jax.ShapeDtypeStruct((Npad, DW), F32)],
    )(hp, s_parts[0], s_parts[1], d_parts[0], d_parts[1], xp16,
      W2, b2r, WnA, WnB, bnr, gr, btr, crwrow)

    return (ho[:N], xo[:N, :3])


# speedup vs baseline: 4.3211x; 2.2446x over previous
import functools

import jax
import jax.numpy as jnp
from jax import lax
from jax.experimental import pallas as pl
from jax.experimental.pallas import tpu as pltpu
from jax.experimental.pallas import tpu_sc as plsc

F32 = jnp.float32
NC, NS, L = 2, 16, 16
BR = 256


def _pre_body(h_ref, w1a_ref, w1b_ref, b1_ref, w2_ref, wct_ref, b2_ref,
              bc_ref, crw_ref, ha_out, hb_out, w2c_out, c_out, crw_out):
    hblk = h_ref[...]
    ha_out[...] = jnp.dot(hblk, w1a_ref[...], precision=lax.Precision.HIGHEST,
                          preferred_element_type=F32) + b1_ref[...]
    hb_out[...] = jnp.dot(hblk, w1b_ref[...], precision=lax.Precision.HIGHEST,
                          preferred_element_type=F32)
    wct = wct_ref[...]
    w2c_out[...] = lax.dot_general(wct, w2_ref[...], (((1,), (1,)), ((), ())),
                                   precision=lax.Precision.HIGHEST,
                                   preferred_element_type=F32)
    cval = lax.dot_general(b2_ref[...], wct, (((1,), (1,)), ((), ())),
                           precision=lax.Precision.HIGHEST,
                           preferred_element_type=F32)
    z = jnp.zeros_like(wct)
    c_out[...] = z + (cval[0, 0] + bc_ref[0, 0])
    crw_out[...] = z + crw_ref[0, 0]


def _post_body(h_ref, s0_ref, s1_ref, d0_ref, d1_ref, x_ref, w2_ref, b2_ref,
               wna_ref, wnb_ref, bn_ref, g_ref, b_ref, crw_ref,
               ho_out, xo_out):
    s = s0_ref[...] + s1_ref[...]
    dsum = d0_ref[...] + d1_ref[...]
    cnt = dsum[:, 3:4]
    agg = jnp.dot(s, w2_ref[...], precision=lax.Precision.HIGHEST,
                  preferred_element_type=F32) + cnt * b2_ref[...]
    hblk = h_ref[...]
    hn = jnp.dot(hblk, wna_ref[...], precision=lax.Precision.HIGHEST,
                 preferred_element_type=F32)
    hn = hn + jnp.dot(agg, wnb_ref[...], precision=lax.Precision.HIGHEST,
                      preferred_element_type=F32) + bn_ref[...]
    hn = jnp.maximum(hn, 0.0)
    y = hblk + hn
    mu = jnp.mean(y, axis=-1, keepdims=True)
    var = jnp.mean((y - mu) ** 2, axis=-1, keepdims=True)
    ho_out[...] = (y - mu) * lax.rsqrt(var + 1e-5) * g_ref[...] + b_ref[...]
    xo_out[...] = x_ref[...] + crw_ref[0, 0] * dsum


def _sc_body(ha_hbm, hb_hbm, x16_hbm, row_hbm, col_hbm, w1c_hbm, w2c_hbm,
             c_hbm, zs_hbm, zd_hbm, s_out, d_out,
             idxr0, idxc0, ha0, hb0, xr0, xc0,
             idxr1, idxc1, ha1, hb1, xr1, xc1,
             d2b, dotb, sbuf, cub, w1cb, w2cb, cvb,
             sem_i0, sem_i1, sem_g0, sem_g1,
             s_sh, d_sh, *, ssh, e_per_tile, ch, dw):
    cid = lax.axis_index("c")
    sid = lax.axis_index("s")
    wid = cid * NS + sid
    rows_per_tile = ssh // NS
    nchunks = e_per_tile // ch

    idxr = [idxr0, idxr1]
    idxc = [idxc0, idxc1]
    ha = [ha0, ha1]
    hb = [hb0, hb1]
    xr16 = [xr0, xr1]
    xc16 = [xc0, xc1]
    sem_i = [sem_i0, sem_i1]
    sem_g = [sem_g0, sem_g1]

    pltpu.sync_copy(w1c_hbm, w1cb)
    pltpu.sync_copy(w2c_hbm.at[0], w2cb)
    pltpu.sync_copy(c_hbm.at[0, pl.ds(0, L)], cvb)

    pltpu.sync_copy(zs_hbm, s_sh.at[pl.ds(sid * rows_per_tile, rows_per_tile)])
    pltpu.sync_copy(zd_hbm, d_sh.at[pl.ds(sid * rows_per_tile, rows_per_tile)])

    lanes = lax.iota(jnp.int32, L)

    plsc.subcore_barrier()

    w1v = [w1cb[pl.ds(L * j, L)] for j in range(8)]
    w2v = [w2cb[pl.ds(L * j, L)] for j in range(8)]
    cvec = cvb[...]
    m15 = lanes == 15

    def ebase(c):
        return wid * e_per_tile + c * ch

    def start_idx(b, c):
        pltpu.async_copy(row_hbm.at[pl.ds(ebase(c), ch)], idxr[b], sem_i[b])
        pltpu.async_copy(col_hbm.at[pl.ds(ebase(c), ch)], idxc[b], sem_i[b])

    def wait_idx(b):
        pltpu.make_async_copy(row_hbm.at[pl.ds(0, ch)], idxr[b],
                              sem_i[b]).wait()
        pltpu.make_async_copy(col_hbm.at[pl.ds(0, ch)], idxc[b],
                              sem_i[b]).wait()

    def start_g(b):
        pltpu.async_copy(ha_hbm.at[idxr[b]], ha[b], sem_g[b])
        pltpu.async_copy(hb_hbm.at[idxc[b]], hb[b], sem_g[b])
        pltpu.async_copy(x16_hbm.at[idxr[b]], xr16[b], sem_g[b])
        pltpu.async_copy(x16_hbm.at[idxc[b]], xc16[b], sem_g[b])

    def wait_g(b):
        pltpu.make_async_copy(ha_hbm.at[idxr[b]], ha[b], sem_g[b]).wait()
        pltpu.make_async_copy(hb_hbm.at[idxc[b]], hb[b], sem_g[b]).wait()
        pltpu.make_async_copy(x16_hbm.at[idxr[b]], xr16[b], sem_g[b]).wait()
        pltpu.make_async_copy(x16_hbm.at[idxc[b]], xc16[b], sem_g[b]).wait()

    def compute(b):
        hab, hbb, xrb, xcb = ha[b], hb[b], xr16[b], xc16[b]

        def _edge(e, _):
            ef = jnp.full((L,), e, jnp.int32)
            rel = xrb[e, :] - xcb[e, :]
            d2c = plsc.cumsum(rel * rel)
            plsc.store_scatter(d2b, [ef], d2c, mask=m15)
            d2s = plsc.load_gather(d2b, [ef])
            acc = jnp.zeros((L,), F32)
            for j in range(8):
                sl = pl.ds(L * j, L)
                t = jnp.maximum(hab[e, sl] + hbb[e, sl] + d2s * w1v[j], 0.0)
                acc = acc + t * w2v[j]
                hab[e, sl] = t
            cs = plsc.cumsum(acc)
            plsc.store_scatter(dotb, [ef], cs, mask=m15)
            return 0
        lax.fori_loop(0, ch, _edge, 0)

        for g in range(ch // L):
            sl = pl.ds(L * g, L)
            zv = jnp.maximum(dotb[sl] + cvec, 0.0)
            e2 = jnp.exp(zv + zv)
            sbuf[sl] = 1.0 - 2.0 / (e2 + 1.0)

        def _cu(e, _):
            ef = jnp.full((L,), e, jnp.int32)
            rel = xrb[e, :] - xcb[e, :]
            ssp = plsc.load_gather(sbuf, [ef])
            cub[e, :] = jnp.where(lanes < 3, rel * ssp,
                                  jnp.where(lanes == 3, 1.0, 0.0))
            return 0
        lax.fori_loop(0, ch, _cu, 0)

        pltpu.sync_copy(hab, s_sh.at[idxr[b]], add=True)
        pltpu.sync_copy(cub, d_sh.at[idxr[b]], add=True)

    start_idx(0, 0)
    wait_idx(0)
    start_g(0)
    start_idx(1, 1)

    def _step(ci, _):
        for b in (0, 1):
            c = 2 * ci + b
            nb = 1 - b
            wait_idx(nb)
            start_g(nb)
            wait_g(b)
            compute(b)
            start_idx(b, c + 2)
        return 0

    lax.fori_loop(0, nchunks // 2, _step, 0)

    wait_g(0)
    wait_idx(1)

    plsc.subcore_barrier()

    ofs = sid * rows_per_tile
    pltpu.sync_copy(s_sh.at[pl.ds(ofs, rows_per_tile)],
                    s_out.at[cid, pl.ds(ofs, rows_per_tile)])
    pltpu.sync_copy(d_sh.at[pl.ds(ofs, rows_per_tile)],
                    d_out.at[cid, pl.ds(ofs, rows_per_tile)])


def kernel(h, x, edge_index, W1, b1, W2, b2, Wc, bc, Wn, bn, gamma, beta, crw):
    N, D = h.shape
    E = edge_index.shape[1]
    assert N == 10000 and D == 128 and E == 320000
    Npad = 10240
    CH = 64
    EPT = 10240
    Epad = EPT * NC * NS
    SSH = 10112
    DW = 16

    hp = jnp.pad(h, ((0, Npad - N), (0, 0)))
    xp16 = jnp.pad(x, ((0, Npad - N), (0, DW - 3)))
    pad_idx = jnp.full((Epad + 2 * CH - E,), N, jnp.int32)
    row = jnp.concatenate([edge_index[0], pad_idx])
    col = jnp.concatenate([edge_index[1], pad_idx])
    W1a, W1b, w1c = W1[:D], W1[D:2 * D], W1[2 * D]
    WnA, WnB = Wn[:D], Wn[D:]
    b1r = b1.reshape(1, D)
    b2r = b2.reshape(1, D)
    bnr = bn.reshape(1, D)
    gr = gamma.reshape(1, D)
    btr = beta.reshape(1, D)
    wct = Wc.reshape(1, D)
    bcr = bc.reshape(1, 1)
    crwr = crw.reshape(1, 1)
    zs = jnp.zeros((SSH // NS, D), F32)
    zd = jnp.zeros((SSH // NS, DW), F32)

    grid = (Npad // BR,)
    full = lambda: pl.BlockSpec(index_map=lambda i: (0, 0))
    rblk = lambda w: pl.BlockSpec((BR, w), lambda i: (i, 0))

    ha_full, hb_full, w2cr, cr, crwrow = pl.pallas_call(
        _pre_body,
        grid=grid,
        in_specs=[rblk(D), full(), full(), full(), full(), full(), full(),
                  full(), full()],
        out_specs=[rblk(D), rblk(D), full(), full(), full()],
        out_shape=[jax.ShapeDtypeStruct((Npad, D), F32),
                   jax.ShapeDtypeStruct((Npad, D), F32),
                   jax.ShapeDtypeStruct((1, D), F32),
                   jax.ShapeDtypeStruct((1, D), F32),
                   jax.ShapeDtypeStruct((1, D), F32)],
    )(hp, W1a, W1b, b1r, W2, wct, b2r, bcr, crwr)

    mesh = plsc.VectorSubcoreMesh(core_axis_name="c", subcore_axis_name="s",
                                  num_cores=NC, num_subcores=NS)
    sc = pl.kernel(
        functools.partial(_sc_body, ssh=SSH, e_per_tile=EPT, ch=CH, dw=DW),
        out_type=[jax.ShapeDtypeStruct((NC, Npad, D), F32),
                  jax.ShapeDtypeStruct((NC, Npad, DW), F32)],
        mesh=mesh,
        compiler_params=pltpu.CompilerParams(use_tc_tiling_on_sc=False,
                                             needs_layout_passes=False),
        scratch_types=(
            [pltpu.VMEM((CH,), jnp.int32),
             pltpu.VMEM((CH,), jnp.int32),
             pltpu.VMEM((CH, D), F32),
             pltpu.VMEM((CH, D), F32),
             pltpu.VMEM((CH, DW), F32),
             pltpu.VMEM((CH, DW), F32)] * 2
            + [pltpu.VMEM((CH,), F32),
               pltpu.VMEM((CH,), F32),
               pltpu.VMEM((CH,), F32),
               pltpu.VMEM((CH, DW), F32),
               pltpu.VMEM((D,), F32),
               pltpu.VMEM((D,), F32),
               pltpu.VMEM((L,), F32)]
            + [pltpu.SemaphoreType.DMA] * 4
            + [pltpu.VMEM_SHARED((SSH, D), F32),
               pltpu.VMEM_SHARED((SSH, DW), F32)]
        ),
    )
    s_parts, d_parts = sc(ha_full, hb_full, xp16, row, col, w1c, w2cr, cr,
                          zs, zd)

    ho, xo = pl.pallas_call(
        _post_body,
        grid=grid,
        in_specs=[rblk(D), rblk(D), rblk(D), rblk(DW), rblk(DW), rblk(DW),
                  full(), full(), full(), full(), full(), full(), full(),
                  full()],
        out_specs=[rblk(D), rblk(DW)],
        out_shape=[jax.ShapeDtypeStruct((Npad, D), F32),
                   jax.ShapeDtypeStruct((Npad, DW), F32)],
    )(hp, s_parts[0], s_parts[1], d_parts[0], d_parts[1], xp16,
      W2, b2r, WnA, WnB, bnr, gr, btr, crwrow)

    return (ho[:N], xo[:N, :3])


# speedup vs baseline: 7.2790x; 1.6845x over previous
import functools

import jax
import jax.numpy as jnp
from jax import lax
from jax.experimental import pallas as pl
from jax.experimental.pallas import tpu as pltpu
from jax.experimental.pallas import tpu_sc as plsc

F32 = jnp.float32
NC, NS, L = 2, 16, 16
BR = 256


def _pre_body(h_ref, w1a_ref, w1b_ref, b1_ref, w2_ref, wct_ref, b2_ref,
              bc_ref, crw_ref, ha_out, hb_out, w2c_out, c_out, crw_out):
    hblk = h_ref[...]
    ha_out[...] = jnp.dot(hblk, w1a_ref[...], precision=lax.Precision.HIGHEST,
                          preferred_element_type=F32) + b1_ref[...]
    hb_out[...] = jnp.dot(hblk, w1b_ref[...], precision=lax.Precision.HIGHEST,
                          preferred_element_type=F32)
    wct = wct_ref[...]
    w2c_out[...] = lax.dot_general(wct, w2_ref[...], (((1,), (1,)), ((), ())),
                                   precision=lax.Precision.HIGHEST,
                                   preferred_element_type=F32)
    cval = lax.dot_general(b2_ref[...], wct, (((1,), (1,)), ((), ())),
                           precision=lax.Precision.HIGHEST,
                           preferred_element_type=F32)
    z = jnp.zeros_like(wct)
    c_out[...] = z + (cval[0, 0] + bc_ref[0, 0])
    crw_out[...] = z + crw_ref[0, 0]


def _post_body(h_ref, s0_ref, s1_ref, d0_ref, d1_ref, x_ref, w2_ref, b2_ref,
               wna_ref, wnb_ref, bn_ref, g_ref, b_ref, crw_ref,
               ho_out, xo_out):
    s = s0_ref[...] + s1_ref[...]
    dsum = d0_ref[...] + d1_ref[...]
    cnt = dsum[:, 3:4]
    agg = jnp.dot(s, w2_ref[...], precision=lax.Precision.HIGHEST,
                  preferred_element_type=F32) + cnt * b2_ref[...]
    hblk = h_ref[...]
    hn = jnp.dot(hblk, wna_ref[...], precision=lax.Precision.HIGHEST,
                 preferred_element_type=F32)
    hn = hn + jnp.dot(agg, wnb_ref[...], precision=lax.Precision.HIGHEST,
                      preferred_element_type=F32) + bn_ref[...]
    hn = jnp.maximum(hn, 0.0)
    y = hblk + hn
    mu = jnp.mean(y, axis=-1, keepdims=True)
    var = jnp.mean((y - mu) ** 2, axis=-1, keepdims=True)
    ho_out[...] = (y - mu) * lax.rsqrt(var + 1e-5) * g_ref[...] + b_ref[...]
    xo_out[...] = x_ref[...] + crw_ref[0, 0] * dsum


def _sc_body(ha_hbm, hb_hbm, x16_hbm, row_hbm, col_hbm, w1c_hbm, w2c_hbm,
             c_hbm, zs_hbm, zd_hbm, s_out, d_out,
             idxr0, idxc0, ha0, hb0, xr0, xc0,
             idxr1, idxc1, ha1, hb1, xr1, xc1,
             d2b, dotb, sbuf, cub, w1cb, w2cb, cvb,
             sem_i0, sem_i1, sem_g0, sem_g1,
             s_sh, d_sh, *, ssh, e_per_tile, ch, dw):
    cid = lax.axis_index("c")
    sid = lax.axis_index("s")
    wid = cid * NS + sid
    rows_per_tile = ssh // NS
    nchunks = e_per_tile // ch

    idxr = [idxr0, idxr1]
    idxc = [idxc0, idxc1]
    ha = [ha0, ha1]
    hb = [hb0, hb1]
    xr16 = [xr0, xr1]
    xc16 = [xc0, xc1]
    sem_i = [sem_i0, sem_i1]
    sem_g = [sem_g0, sem_g1]

    pltpu.sync_copy(w1c_hbm, w1cb)
    pltpu.sync_copy(w2c_hbm.at[0], w2cb)
    pltpu.sync_copy(c_hbm.at[0, pl.ds(0, L)], cvb)

    pltpu.sync_copy(zs_hbm, s_sh.at[pl.ds(sid * rows_per_tile, rows_per_tile)])
    pltpu.sync_copy(zd_hbm, d_sh.at[pl.ds(sid * rows_per_tile, rows_per_tile)])

    lanes = lax.iota(jnp.int32, L)

    plsc.subcore_barrier()

    w1v = [w1cb[pl.ds(L * j, L)] for j in range(8)]
    w2v = [w2cb[pl.ds(L * j, L)] for j in range(8)]
    cvec = cvb[...]
    m15 = lanes == 15

    def ebase(c):
        return wid * e_per_tile + c * ch

    def start_idx(b, c):
        pltpu.async_copy(row_hbm.at[pl.ds(ebase(c), ch)], idxr[b], sem_i[b])
        pltpu.async_copy(col_hbm.at[pl.ds(ebase(c), ch)], idxc[b], sem_i[b])

    def wait_idx(b):
        pltpu.make_async_copy(row_hbm.at[pl.ds(0, ch)], idxr[b],
                              sem_i[b]).wait()
        pltpu.make_async_copy(col_hbm.at[pl.ds(0, ch)], idxc[b],
                              sem_i[b]).wait()

    def start_g(b):
        pltpu.async_copy(ha_hbm.at[idxr[b]], ha[b], sem_g[b])
        pltpu.async_copy(hb_hbm.at[idxc[b]], hb[b], sem_g[b])
        pltpu.async_copy(x16_hbm.at[idxr[b]], xr16[b], sem_g[b])
        pltpu.async_copy(x16_hbm.at[idxc[b]], xc16[b], sem_g[b])

    def wait_g(b):
        pltpu.make_async_copy(ha_hbm.at[idxr[b]], ha[b], sem_g[b]).wait()
        pltpu.make_async_copy(hb_hbm.at[idxc[b]], hb[b], sem_g[b]).wait()
        pltpu.make_async_copy(x16_hbm.at[idxr[b]], xr16[b], sem_g[b]).wait()
        pltpu.make_async_copy(x16_hbm.at[idxc[b]], xc16[b], sem_g[b]).wait()

    def compute(b):
        hab, hbb, xrb, xcb = ha[b], hb[b], xr16[b], xc16[b]

        @plsc.parallel_loop(0, ch, unroll=2)
        def _edge(e):
            ef = jnp.full((L,), e, jnp.int32)
            rel = xrb[e, :] - xcb[e, :]
            d2c = plsc.cumsum(rel * rel)
            plsc.store_scatter(d2b, [ef], d2c, mask=m15)
            d2s = plsc.load_gather(d2b, [ef])
            acc = jnp.zeros((L,), F32)
            for j in range(8):
                sl = pl.ds(L * j, L)
                t = jnp.maximum(hab[e, sl] + hbb[e, sl] + d2s * w1v[j], 0.0)
                acc = acc + t * w2v[j]
                hab[e, sl] = t
            cs = plsc.cumsum(acc)
            zv = jnp.maximum(cs + cvec, 0.0)
            e2 = jnp.exp(zv + zv)
            sv = 1.0 - 2.0 / (e2 + 1.0)
            plsc.store_scatter(sbuf, [ef], sv, mask=m15)
            ssp = plsc.load_gather(sbuf, [ef])
            cub[e, :] = jnp.where(lanes < 3, rel * ssp,
                                  jnp.where(lanes == 3, 1.0, 0.0))

        pltpu.sync_copy(hab, s_sh.at[idxr[b]], add=True)
        pltpu.sync_copy(cub, d_sh.at[idxr[b]], add=True)

    start_idx(0, 0)
    wait_idx(0)
    start_g(0)
    start_idx(1, 1)

    def _step(ci, _):
        for b in (0, 1):
            c = 2 * ci + b
            nb = 1 - b
            wait_idx(nb)
            start_g(nb)
            wait_g(b)
            compute(b)
            start_idx(b, c + 2)
        return 0

    lax.fori_loop(0, nchunks // 2, _step, 0)

    wait_g(0)
    wait_idx(1)

    plsc.subcore_barrier()

    ofs = sid * rows_per_tile
    pltpu.sync_copy(s_sh.at[pl.ds(ofs, rows_per_tile)],
                    s_out.at[cid, pl.ds(ofs, rows_per_tile)])
    pltpu.sync_copy(d_sh.at[pl.ds(ofs, rows_per_tile)],
                    d_out.at[cid, pl.ds(ofs, rows_per_tile)])


def kernel(h, x, edge_index, W1, b1, W2, b2, Wc, bc, Wn, bn, gamma, beta, crw):
    N, D = h.shape
    E = edge_index.shape[1]
    assert N == 10000 and D == 128 and E == 320000
    Npad = 10240
    CH = 64
    EPT = 10240
    Epad = EPT * NC * NS
    SSH = 10112
    DW = 16

    hp = jnp.pad(h, ((0, Npad - N), (0, 0)))
    xp16 = jnp.pad(x, ((0, Npad - N), (0, DW - 3)))
    pad_idx = jnp.full((Epad + 2 * CH - E,), N, jnp.int32)
    row = jnp.concatenate([edge_index[0], pad_idx])
    col = jnp.concatenate([edge_index[1], pad_idx])
    W1a, W1b, w1c = W1[:D], W1[D:2 * D], W1[2 * D]
    WnA, WnB = Wn[:D], Wn[D:]
    b1r = b1.reshape(1, D)
    b2r = b2.reshape(1, D)
    bnr = bn.reshape(1, D)
    gr = gamma.reshape(1, D)
    btr = beta.reshape(1, D)
    wct = Wc.reshape(1, D)
    bcr = bc.reshape(1, 1)
    crwr = crw.reshape(1, 1)
    zs = jnp.zeros((SSH // NS, D), F32)
    zd = jnp.zeros((SSH // NS, DW), F32)

    grid = (Npad // BR,)
    full = lambda: pl.BlockSpec(index_map=lambda i: (0, 0))
    rblk = lambda w: pl.BlockSpec((BR, w), lambda i: (i, 0))

    ha_full, hb_full, w2cr, cr, crwrow = pl.pallas_call(
        _pre_body,
        grid=grid,
        in_specs=[rblk(D), full(), full(), full(), full(), full(), full(),
                  full(), full()],
        out_specs=[rblk(D), rblk(D), full(), full(), full()],
        out_shape=[jax.ShapeDtypeStruct((Npad, D), F32),
                   jax.ShapeDtypeStruct((Npad, D), F32),
                   jax.ShapeDtypeStruct((1, D), F32),
                   jax.ShapeDtypeStruct((1, D), F32),
                   jax.ShapeDtypeStruct((1, D), F32)],
    )(hp, W1a, W1b, b1r, W2, wct, b2r, bcr, crwr)

    mesh = plsc.VectorSubcoreMesh(core_axis_name="c", subcore_axis_name="s",
                                  num_cores=NC, num_subcores=NS)
    sc = pl.kernel(
        functools.partial(_sc_body, ssh=SSH, e_per_tile=EPT, ch=CH, dw=DW),
        out_type=[jax.ShapeDtypeStruct((NC, Npad, D), F32),
                  jax.ShapeDtypeStruct((NC, Npad, DW), F32)],
        mesh=mesh,
        compiler_params=pltpu.CompilerParams(use_tc_tiling_on_sc=False,
                                             needs_layout_passes=False),
        scratch_types=(
            [pltpu.VMEM((CH,), jnp.int32),
             pltpu.VMEM((CH,), jnp.int32),
             pltpu.VMEM((CH, D), F32),
             pltpu.VMEM((CH, D), F32),
             pltpu.VMEM((CH, DW), F32),
             pltpu.VMEM((CH, DW), F32)] * 2
            + [pltpu.VMEM((CH,), F32),
               pltpu.VMEM((CH,), F32),
               pltpu.VMEM((CH,), F32),
               pltpu.VMEM((CH, DW), F32),
               pltpu.VMEM((D,), F32),
               pltpu.VMEM((D,), F32),
               pltpu.VMEM((L,), F32)]
            + [pltpu.SemaphoreType.DMA] * 4
            + [pltpu.VMEM_SHARED((SSH, D), F32),
               pltpu.VMEM_SHARED((SSH, DW), F32)]
        ),
    )
    s_parts, d_parts = sc(ha_full, hb_full, xp16, row, col, w1c, w2cr, cr,
                          zs, zd)

    ho, xo = pl.pallas_call(
        _post_body,
        grid=grid,
        in_specs=[rblk(D), rblk(D), rblk(D), rblk(DW), rblk(DW), rblk(DW),
                  full(), full(), full(), full(), full(), full(), full(),
                  full()],
        out_specs=[rblk(D), rblk(DW)],
        out_shape=[jax.ShapeDtypeStruct((Npad, D), F32),
                   jax.ShapeDtypeStruct((Npad, DW), F32)],
    )(hp, s_parts[0], s_parts[1], d_parts[0], d_parts[1], xp16,
      W2, b2r, WnA, WnB, bnr, gr, btr, crwrow)

    return (ho[:N], xo[:N, :3])
